# baseline (speedup 1.0000x reference)
import jax
import jax.numpy as jnp
from jax import lax
from jax.experimental import pallas as pl
from jax.experimental.pallas import tpu as pltpu
from jax.experimental.pallas import tpu_sc as plsc

_SLOT_ROWS = (64, 56)


def _chunk_schedule(rows_per_w):
    sched = []
    off = 0
    i = 0
    while off < rows_per_w:
        slot = i % 2
        sz = min(_SLOT_ROWS[slot], rows_per_w - off)
        sched.append((off, sz, slot))
        off += sz
        i += 1
    return sched


def _make_sc_kernel(B, S, H, dtype):
    info = plsc.get_sparse_core_info()
    NC, NS = info.num_cores, info.num_subcores
    NW = NC * NS
    rows_per_w = S // NW
    sched = _chunk_schedule(rows_per_w)
    n = len(sched)
    mesh = plsc.VectorSubcoreMesh(core_axis_name="c", subcore_axis_name="s")

    def body(pos_hbm, out_hbm, vbuf_a, vbuf_b, in_sem, out_sem):
        bufs = (vbuf_a, vbuf_b)
        wid = lax.axis_index("s") * NC + lax.axis_index("c")
        base = wid * rows_per_w

        def in_cp(i):
            off, sz, slot = sched[i]
            return pltpu.make_async_copy(
                pos_hbm.at[pl.ds(base + off, sz), :],
                bufs[slot].at[pl.ds(0, sz), :],
                in_sem.at[slot],
            )

        def out_cp(i, b):
            off, sz, slot = sched[i]
            return pltpu.make_async_copy(
                bufs[slot].at[pl.ds(0, sz), :],
                out_hbm.at[b, pl.ds(base + off, sz), :],
                out_sem.at[slot],
            )

        in_cp(0).start()
        for i in range(n):
            in_cp(i).wait()
            for b in range(B):
                out_cp(i, b).start()
            if i + 1 < n:
                if i >= 1:
                    for b in range(B):
                        out_cp(i - 1, b).wait()
                in_cp(i + 1).start()
        for i in (n - 2, n - 1):
            for b in range(B):
                out_cp(i, b).wait()

    return pl.kernel(
        body,
        out_type=jax.ShapeDtypeStruct((B, S, H), dtype),
        mesh=mesh,
        scratch_types=[
            pltpu.VMEM((_SLOT_ROWS[0], H), dtype),
            pltpu.VMEM((_SLOT_ROWS[1], H), dtype),
            pltpu.SemaphoreType.DMA((2,)),
            pltpu.SemaphoreType.DMA((2,)),
        ],
    )


def kernel(x, pos_emb):
    B, S = x.shape
    N, H = pos_emb.shape
    return _make_sc_kernel(B, S, H, pos_emb.dtype)(pos_emb)


# speedup vs baseline: 1.0083x; 1.0083x over previous
import jax
import jax.numpy as jnp
from jax import lax
from jax.experimental import pallas as pl
from jax.experimental.pallas import tpu as pltpu
from jax.experimental.pallas import tpu_sc as plsc

_SLOT_ROWS = (64, 56)


def _chunk_schedule(rows_per_w):
    sched = []
    off = 0
    i = 0
    while off < rows_per_w:
        slot = i % 2
        sz = min(_SLOT_ROWS[slot], rows_per_w - off)
        sched.append((off, sz, slot))
        off += sz
        i += 1
    return sched


def _make_sc_kernel(B, S, H, dtype):
    info = plsc.get_sparse_core_info()
    NC, NS = info.num_cores, info.num_subcores
    NW = NC * NS
    rows_per_w = S // NW
    sched = _chunk_schedule(rows_per_w)
    n = len(sched)
    mesh = plsc.VectorSubcoreMesh(core_axis_name="c", subcore_axis_name="s")

    def body(pos_hbm, out_hbm, vbuf_a, vbuf_b, in_sem, out_sem):
        bufs = (vbuf_a, vbuf_b)
        wid = lax.axis_index("s") * NC + lax.axis_index("c")
        base = wid * rows_per_w

        def in_cp(i):
            off, sz, slot = sched[i]
            return pltpu.make_async_copy(
                pos_hbm.at[pl.ds(base + off, sz), :],
                bufs[slot].at[pl.ds(0, sz), :],
                in_sem.at[slot],
            )

        def out_cp(i, b):
            off, sz, slot = sched[i]
            return pltpu.make_async_copy(
                bufs[slot].at[pl.ds(0, sz), :],
                out_hbm.at[b, pl.ds(base + off, sz), :],
                out_sem.at[slot],
            )

        in_cp(0).start()
        for i in range(n):
            in_cp(i).wait()
            if i + 1 < n:
                if i >= 1:
                    for b in range(B):
                        out_cp(i - 1, b).wait()
                in_cp(i + 1).start()
            for b in range(B):
                out_cp(i, b).start()
        for i in (n - 2, n - 1):
            for b in range(B):
                out_cp(i, b).wait()

    return pl.kernel(
        body,
        out_type=jax.ShapeDtypeStruct((B, S, H), dtype),
        mesh=mesh,
        scratch_types=[
            pltpu.VMEM((_SLOT_ROWS[0], H), dtype),
            pltpu.VMEM((_SLOT_ROWS[1], H), dtype),
            pltpu.SemaphoreType.DMA((2,)),
            pltpu.SemaphoreType.DMA((2,)),
        ],
    )


def kernel(x, pos_emb):
    B, S = x.shape
    N, H = pos_emb.shape
    return _make_sc_kernel(B, S, H, pos_emb.dtype)(pos_emb)
